# baseline (speedup 1.0000x reference)
import functools

import jax
import jax.numpy as jnp
from jax import lax
from jax.experimental import pallas as pl
from jax.experimental.pallas import tpu as pltpu
from jax.experimental.pallas import tpu_sc as plsc

BATCH = 16384
EMB = 64
L = 16
NC, NS = 2, 16
NW = NC * NS
BPW = BATCH // NW
NG = BPW // L

S1 = 33
S2 = 257
ST = 17


def _tt_body(x_hbm, g1_hbm, g2_hbm, g3_hbm, out_hbm, t1, t2, t3, xv, outv, tp):
    cid = lax.axis_index("c")
    sid = lax.axis_index("s")
    wid = sid * NC + cid
    base = wid * BPW

    pltpu.sync_copy(g1_hbm, t1)
    pltpu.sync_copy(g2_hbm, t2)
    pltpu.sync_copy(g3_hbm, t3)
    pltpu.sync_copy(x_hbm.at[pl.ds(base, BPW)], xv)

    iota = lax.iota(jnp.int32, L)
    iota_t = iota * ST

    def group(g, carry):
        x16 = xv[pl.ds(g * L, L)]
        i1 = x16 // 10000
        i2 = (x16 // 100) % 100
        i3 = x16 % 100
        b1 = i1 * S1
        b2 = i2 * S2
        b3 = i3 * S1

        a1 = [[plsc.load_gather(t1, [b1 + (a * 8 + p)]) for p in range(8)]
              for a in range(4)]
        a3 = [[plsc.load_gather(t3, [b3 + (q * 4 + d)]) for d in range(4)]
              for q in range(8)]

        for c in range(4):
            sv = [[None] * 4 for _ in range(8)]
            for p in range(8):
                for q in range(8):
                    gv = plsc.load_gather(t2, [b2 + (p * 32 + c * 8 + q)])
                    for d in range(4):
                        term = gv * a3[q][d]
                        sv[p][d] = term if sv[p][d] is None else sv[p][d] + term
            for a in range(4):
                for d in range(4):
                    acc = None
                    for p in range(8):
                        term = a1[a][p] * sv[p][d]
                        acc = term if acc is None else acc + term
                    e = a * 16 + c * 4 + d
                    tp[pl.ds(e * ST, L)] = acc
        for r in range(L):
            for k in range(4):
                vec = plsc.load_gather(tp, [iota_t + (272 * k + r)])
                outv[g * L + r, pl.ds(16 * k, L)] = vec
        return carry

    lax.fori_loop(0, NG, group, 0)

    pltpu.sync_copy(outv, out_hbm.at[pl.ds(base, BPW)])


@jax.jit
def kernel(x, G1, G2, G3):
    xf = x.reshape(-1).astype(jnp.int32)
    t1 = jnp.pad(G1[0].reshape(100, 32), ((0, 0), (0, S1 - 32)))
    t2 = jnp.pad(jnp.transpose(G2, (1, 0, 2, 3)).reshape(100, 256),
                 ((0, 0), (0, S2 - 256)))
    t3 = jnp.pad(jnp.transpose(G3[..., 0], (1, 0, 2)).reshape(100, 32),
                 ((0, 0), (0, S1 - 32)))
    mesh = plsc.VectorSubcoreMesh(core_axis_name="c", subcore_axis_name="s")
    run = functools.partial(
        pl.kernel,
        mesh=mesh,
        out_type=jax.ShapeDtypeStruct((BATCH, EMB), jnp.float32),
        scratch_types=[
            pltpu.VMEM((100 * S1,), jnp.float32),
            pltpu.VMEM((100 * S2,), jnp.float32),
            pltpu.VMEM((100 * S1,), jnp.float32),
            pltpu.VMEM((BPW,), jnp.int32),
            pltpu.VMEM((BPW, EMB), jnp.float32),
            pltpu.VMEM((EMB * ST,), jnp.float32),
        ],
        compiler_params=pltpu.CompilerParams(needs_layout_passes=False),
    )(_tt_body)
    rows = run(xf, t1.reshape(-1), t2.reshape(-1), t3.reshape(-1))
    return rows.reshape(list(x.shape) + [EMB])


# speedup vs baseline: 2.6683x; 2.6683x over previous
import functools

import jax
import jax.numpy as jnp
from jax import lax
from jax.experimental import pallas as pl
from jax.experimental.pallas import tpu as pltpu
from jax.experimental.pallas import tpu_sc as plsc

BATCH = 16384
EMB = 64
L = 16
NC, NS = 2, 16
NW = NC * NS
BPW = BATCH // NW
NG = BPW // L

S1 = 33
S2 = 257
SO = 65


def _tt_body(x_hbm, g1_hbm, g2_hbm, g3_hbm, out_hbm, t1, t2, t3, xv, outv):
    cid = lax.axis_index("c")
    sid = lax.axis_index("s")
    wid = sid * NC + cid
    base = wid * BPW

    pltpu.sync_copy(g1_hbm, t1)
    pltpu.sync_copy(g2_hbm, t2)
    pltpu.sync_copy(g3_hbm, t3)
    pltpu.sync_copy(x_hbm.at[pl.ds(base, BPW)], xv)

    iota = lax.iota(jnp.int32, L)
    cols = [iota * 0 + e for e in range(EMB)]

    @functools.partial(plsc.parallel_loop, 0, NG)
    def group(g):
        x16 = xv[pl.ds(g * L, L)]
        i1 = x16 // 10000
        i2 = (x16 // 100) % 100
        i3 = x16 % 100
        b1 = i1 * S1
        b2 = i2 * S2
        b3 = i3 * S1
        row = g * L + iota

        a1 = [[plsc.load_gather(t1, [b1 + (a * 8 + p)]) for p in range(8)]
              for a in range(4)]
        a3 = [[plsc.load_gather(t3, [b3 + (q * 4 + d)]) for d in range(4)]
              for q in range(8)]

        for c in range(4):
            sv = [[None] * 4 for _ in range(8)]
            for p in range(8):
                for q in range(8):
                    gv = plsc.load_gather(t2, [b2 + (p * 32 + c * 8 + q)])
                    for d in range(4):
                        term = gv * a3[q][d]
                        sv[p][d] = term if sv[p][d] is None else sv[p][d] + term
            for a in range(4):
                for d in range(4):
                    acc = None
                    for p in range(8):
                        term = a1[a][p] * sv[p][d]
                        acc = term if acc is None else acc + term
                    e = a * 16 + c * 4 + d
                    plsc.store_scatter(outv, [row, cols[e]], acc)

    pltpu.sync_copy(outv, out_hbm.at[pl.ds(base, BPW)])


@jax.jit
def kernel(x, G1, G2, G3):
    xf = x.reshape(-1).astype(jnp.int32)
    t1 = jnp.pad(G1[0].reshape(100, 32), ((0, 0), (0, S1 - 32)))
    t2 = jnp.pad(jnp.transpose(G2, (1, 0, 2, 3)).reshape(100, 256),
                 ((0, 0), (0, S2 - 256)))
    t3 = jnp.pad(jnp.transpose(G3[..., 0], (1, 0, 2)).reshape(100, 32),
                 ((0, 0), (0, S1 - 32)))
    mesh = plsc.VectorSubcoreMesh(core_axis_name="c", subcore_axis_name="s")
    run = functools.partial(
        pl.kernel,
        mesh=mesh,
        out_type=jax.ShapeDtypeStruct((BATCH, SO), jnp.float32),
        scratch_types=[
            pltpu.VMEM((100 * S1,), jnp.float32),
            pltpu.VMEM((100 * S2,), jnp.float32),
            pltpu.VMEM((100 * S1,), jnp.float32),
            pltpu.VMEM((BPW,), jnp.int32),
            pltpu.VMEM((BPW, SO), jnp.float32),
        ],
        compiler_params=pltpu.CompilerParams(needs_layout_passes=False),
    )(_tt_body)
    rows = run(xf, t1.reshape(-1), t2.reshape(-1), t3.reshape(-1))
    return rows[:, :EMB].reshape(list(x.shape) + [EMB])
